# baseline (speedup 1.0000x reference)
import functools

import jax
import jax.numpy as jnp
from jax import lax
from jax.experimental import pallas as pl
from jax.experimental.pallas import tpu as pltpu
from jax.experimental.pallas import tpu_sc as plsc

NUM_E = 8192
DIM = 256
N_TOK = 16384
BM = 1024
BN = 1024
RW0, RW1 = 342, 684
NR = NUM_E // 8


def _bf16_rne(x):
    b = lax.bitcast_convert_type(x, jnp.uint32)
    r = (b + jnp.uint32(0x7FFF) + ((b >> 16) & jnp.uint32(1))) & jnp.uint32(0xFFFF0000)
    return lax.bitcast_convert_type(r, jnp.float32)


def _dist_argmin_body(r1_ref, r2_ref, z_ref, cb_ref, idx_ref, dpick_ref, m_s):
    zb2 = z_ref[...] * 2.0
    r1v = r1_ref[...]

    inf8 = jnp.full((8, BM), jnp.inf, jnp.float32)
    zero8 = jnp.zeros((8, BM), jnp.int32)
    wins = [(inf8, zero8), (inf8, zero8), (inf8, zero8)]

    nchunks = NUM_E // BN
    rpc = BN // 8

    for c in range(nchunks):
        m_s[...] = lax.dot_general(
            cb_ref[pl.ds(c * BN, BN), :], zb2, (((1,), (1,)), ((), ())),
            preferred_element_type=jnp.float32)

        lo = c * rpc
        segs = []
        for w, (wlo, whi) in enumerate(((0, RW0), (RW0, RW1), (RW1, NR))):
            slo, shi = max(lo, wlo), min(lo + rpc, whi)
            if slo < shi:
                segs.append((w, slo - lo, shi - lo))

        for w, rlo, rhi in segs:
            def step(k, carry, _rlo=rlo, _lo=lo):
                bv, bi = carry
                rloc = _rlo + k
                ms = m_s[pl.ds(rloc * 8, 8), :]
                r2s = r2_ref[pl.ds((_lo + rloc) * 8, 8), :]
                dv = (r1v + r2s) - ms
                take = dv < bv
                bv = jnp.where(take, dv, bv)
                bi = jnp.where(take, jnp.full((8, BM), 0, jnp.int32) + (_lo + rloc), bi)
                return bv, bi
            wins[w] = lax.fori_loop(0, rhi - rlo, step, wins[w])

    s_iota = lax.broadcasted_iota(jnp.int32, (8, BM), 0)

    def smerge(vals, js):
        js = js * 8 + s_iota
        for h in (4, 2, 1):
            v1, v2 = vals[0:h], vals[h:2 * h]
            j1, j2 = js[0:h], js[h:2 * h]
            take = (v2 < v1) | ((v2 == v1) & (j2 < j1))
            vals = jnp.where(take, v2, v1)
            js = jnp.where(take, j2, j1)
        return vals, js

    (v0, i0) = smerge(*wins[0])
    (v1, i1) = smerge(*wins[1])
    (v2, i2) = smerge(*wins[2])

    accv = _bf16_rne(v0)
    acci = i0
    pickv = v0
    take = v1 < accv
    accv = jnp.where(take, _bf16_rne(v1), accv)
    acci = jnp.where(take, i1, acci)
    pickv = jnp.where(take, v1, pickv)
    take = v2 < accv
    acci = jnp.where(take, i2, acci)
    pickv = jnp.where(take, v2, pickv)

    idx_ref[...] = acci.reshape(BM)
    dpick_ref[...] = pickv.reshape(BM)


def _dist_argmin(r1, r2, z_flat, codebook):
    return pl.pallas_call(
        _dist_argmin_body,
        grid=(N_TOK // BM,),
        in_specs=[
            pl.BlockSpec((1, BM), lambda i: (0, i)),
            pl.BlockSpec((NUM_E, 1), lambda i: (0, 0)),
            pl.BlockSpec((BM, DIM), lambda i: (i, 0)),
            pl.BlockSpec((NUM_E, DIM), lambda i: (0, 0)),
        ],
        out_specs=[
            pl.BlockSpec((BM,), lambda i: (i,)),
            pl.BlockSpec((BM,), lambda i: (i,)),
        ],
        out_shape=[
            jax.ShapeDtypeStruct((N_TOK,), jnp.int32),
            jax.ShapeDtypeStruct((N_TOK,), jnp.float32),
        ],
        scratch_shapes=[pltpu.VMEM((BN, BM), jnp.float32)],
    )(r1, r2, z_flat, codebook)


_SC_CHUNK = 128


def _sc_gather(codebook, idx):
    info = plsc.get_sparse_core_info()
    nw = info.num_cores * info.num_subcores
    b_per_w = N_TOK // nw
    nchunk = b_per_w // _SC_CHUNK
    mesh = plsc.VectorSubcoreMesh(core_axis_name="c", subcore_axis_name="s")

    @functools.partial(
        pl.kernel, mesh=mesh,
        out_type=jax.ShapeDtypeStruct((N_TOK, DIM), jnp.float32),
        scratch_types=[
            pltpu.VMEM((_SC_CHUNK,), jnp.int32),
            pltpu.VMEM((_SC_CHUNK, DIM), jnp.float32),
            pltpu.SemaphoreType.DMA,
        ],
    )
    def gather(table_hbm, idx_hbm, out_hbm, idx_v, rows_v, sem):
        wid = lax.axis_index("s") * info.num_cores + lax.axis_index("c")
        base = wid * b_per_w
        for t in range(nchunk):
            off = base + t * _SC_CHUNK
            pltpu.sync_copy(idx_hbm.at[pl.ds(off, _SC_CHUNK)], idx_v)
            pltpu.async_copy(table_hbm.at[idx_v], rows_v, sem).wait()
            pltpu.sync_copy(rows_v, out_hbm.at[pl.ds(off, _SC_CHUNK)])

    return gather(codebook, idx)


def kernel(z, codebook):
    zz = z[0]
    z_flat = zz.reshape(-1, DIM)
    r1 = jnp.sum(zz ** 2, axis=-1).reshape(1, -1)
    r2 = jnp.sum(codebook ** 2, axis=1).reshape(-1, 1)
    idx, dpick = _dist_argmin(r1, r2, z_flat, codebook)
    zq = _sc_gather(codebook, idx)
    z_out = zq.reshape(zz.shape)
    vq_loss = 1.1 * (jnp.sum(dpick) / jnp.float32(z_flat.size))
    return (z_out, vq_loss)


# speedup vs baseline: 6.2694x; 6.2694x over previous
import functools

import jax
import jax.numpy as jnp
from jax import lax
from jax.experimental import pallas as pl
from jax.experimental.pallas import tpu as pltpu
from jax.experimental.pallas import tpu_sc as plsc

NUM_E = 8192
DIM = 256
N_TOK = 16384
BM = 1024
BN = 1024
RW0, RW1 = 342, 684
NR = NUM_E // 8


def _bf16_rne(x):
    b = lax.bitcast_convert_type(x, jnp.uint32)
    r = (b + jnp.uint32(0x7FFF) + ((b >> 16) & jnp.uint32(1))) & jnp.uint32(0xFFFF0000)
    return lax.bitcast_convert_type(r, jnp.float32)


def _dist_argmin_body(r1_ref, r2_ref, z_ref, cb_ref, idx_ref, dpick_ref, m_s0, m_s1):
    zb2 = z_ref[...] * 2.0
    r1v = r1_ref[...]

    inf8 = jnp.full((8, BM), jnp.inf, jnp.float32)
    zero8 = jnp.zeros((8, BM), jnp.int32)
    wins = [(inf8, zero8), (inf8, zero8), (inf8, zero8)]

    nchunks = NUM_E // BN
    rpc = BN // 8
    bufs = (m_s0, m_s1)

    for c in range(nchunks):
        m_s = bufs[c % 2]
        m_s[...] = lax.dot_general(
            cb_ref[pl.ds(c * BN, BN), :], zb2, (((1,), (1,)), ((), ())),
            preferred_element_type=jnp.float32)
        r2c = r2_ref[pl.ds(c * BN, BN), :]
        d3 = ((r1v + r2c) - m_s[...]).reshape(rpc, 8, BM)

        lo = c * rpc
        for w, (wlo, whi) in enumerate(((0, RW0), (RW0, RW1), (RW1, NR))):
            slo, shi = max(lo, wlo), min(lo + rpc, whi)
            if slo >= shi:
                continue
            seg = d3[slo - lo:shi - lo]
            nr = shi - slo
            segmin = jnp.min(seg, axis=0)
            riota = lax.broadcasted_iota(jnp.int32, (nr, 8, BM), 0) + slo
            cand = jnp.where(seg == segmin[None], riota, 2**30)
            segidx = jnp.min(cand, axis=0)
            bv, bi = wins[w]
            take = (segmin < bv) | ((segmin == bv) & (segidx < bi))
            wins[w] = (jnp.where(take, segmin, bv), jnp.where(take, segidx, bi))

    s_iota = lax.broadcasted_iota(jnp.int32, (8, BM), 0)

    def smerge(vals, js):
        js = js * 8 + s_iota
        for h in (4, 2, 1):
            v1, v2 = vals[0:h], vals[h:2 * h]
            j1, j2 = js[0:h], js[h:2 * h]
            take = (v2 < v1) | ((v2 == v1) & (j2 < j1))
            vals = jnp.where(take, v2, v1)
            js = jnp.where(take, j2, j1)
        return vals, js

    (v0, i0) = smerge(*wins[0])
    (v1, i1) = smerge(*wins[1])
    (v2, i2) = smerge(*wins[2])

    accv = _bf16_rne(v0)
    acci = i0
    pickv = v0
    take = v1 < accv
    accv = jnp.where(take, _bf16_rne(v1), accv)
    acci = jnp.where(take, i1, acci)
    pickv = jnp.where(take, v1, pickv)
    take = v2 < accv
    acci = jnp.where(take, i2, acci)
    pickv = jnp.where(take, v2, pickv)

    idx_ref[...] = acci.reshape(BM)
    dpick_ref[...] = pickv.reshape(BM)


def _dist_argmin(r1, r2, z_flat, codebook):
    return pl.pallas_call(
        _dist_argmin_body,
        grid=(N_TOK // BM,),
        in_specs=[
            pl.BlockSpec((1, BM), lambda i: (0, i)),
            pl.BlockSpec((NUM_E, 1), lambda i: (0, 0)),
            pl.BlockSpec((BM, DIM), lambda i: (i, 0)),
            pl.BlockSpec((NUM_E, DIM), lambda i: (0, 0)),
        ],
        out_specs=[
            pl.BlockSpec((BM,), lambda i: (i,)),
            pl.BlockSpec((BM,), lambda i: (i,)),
        ],
        out_shape=[
            jax.ShapeDtypeStruct((N_TOK,), jnp.int32),
            jax.ShapeDtypeStruct((N_TOK,), jnp.float32),
        ],
        scratch_shapes=[pltpu.VMEM((BN, BM), jnp.float32),
                        pltpu.VMEM((BN, BM), jnp.float32)],
    )(r1, r2, z_flat, codebook)


_SC_CHUNK = 128


def _sc_gather(codebook, idx):
    info = plsc.get_sparse_core_info()
    nw = info.num_cores * info.num_subcores
    b_per_w = N_TOK // nw
    nchunk = b_per_w // _SC_CHUNK
    mesh = plsc.VectorSubcoreMesh(core_axis_name="c", subcore_axis_name="s")

    @functools.partial(
        pl.kernel, mesh=mesh,
        out_type=jax.ShapeDtypeStruct((N_TOK, DIM), jnp.float32),
        scratch_types=[
            pltpu.VMEM((_SC_CHUNK,), jnp.int32),
            pltpu.VMEM((_SC_CHUNK, DIM), jnp.float32),
            pltpu.SemaphoreType.DMA,
        ],
    )
    def gather(table_hbm, idx_hbm, out_hbm, idx_v, rows_v, sem):
        wid = lax.axis_index("s") * info.num_cores + lax.axis_index("c")
        base = wid * b_per_w
        for t in range(nchunk):
            off = base + t * _SC_CHUNK
            pltpu.sync_copy(idx_hbm.at[pl.ds(off, _SC_CHUNK)], idx_v)
            pltpu.async_copy(table_hbm.at[idx_v], rows_v, sem).wait()
            pltpu.sync_copy(rows_v, out_hbm.at[pl.ds(off, _SC_CHUNK)])

    return gather(codebook, idx)


def kernel(z, codebook):
    zz = z[0]
    z_flat = zz.reshape(-1, DIM)
    r1 = jnp.sum(zz ** 2, axis=-1).reshape(1, -1)
    r2 = jnp.sum(codebook ** 2, axis=1).reshape(-1, 1)
    idx, dpick = _dist_argmin(r1, r2, z_flat, codebook)
    zq = _sc_gather(codebook, idx)
    z_out = zq.reshape(zz.shape)
    vq_loss = 1.1 * (jnp.sum(dpick) / jnp.float32(z_flat.size))
    return (z_out, vq_loss)


# speedup vs baseline: 6.8271x; 1.0890x over previous
import functools

import jax
import jax.numpy as jnp
from jax import lax
from jax.experimental import pallas as pl
from jax.experimental.pallas import tpu as pltpu
from jax.experimental.pallas import tpu_sc as plsc

NUM_E = 8192
DIM = 256
N_TOK = 16384
BM = 1024
BN = 1024
RW0, RW1 = 342, 684
NR = NUM_E // 8


def _bf16_rne(x):
    b = lax.bitcast_convert_type(x, jnp.uint32)
    r = (b + jnp.uint32(0x7FFF) + ((b >> 16) & jnp.uint32(1))) & jnp.uint32(0xFFFF0000)
    return lax.bitcast_convert_type(r, jnp.float32)


def _dist_argmin_body(r1_ref, r2_ref, z_ref, cb_ref, idx_ref, dpick_ref, m_s0, m_s1):
    zb2 = z_ref[...] * 2.0
    r1v = r1_ref[...]

    inf8 = jnp.full((8, BM), jnp.inf, jnp.float32)
    zero8 = jnp.zeros((8, BM), jnp.float32)
    wins = [(inf8, zero8), (inf8, zero8), (inf8, zero8)]

    nchunks = NUM_E // BN
    rpc = BN // 8
    bufs = (m_s0, m_s1)

    for c in range(nchunks):
        m_s = bufs[c % 2]
        m_s[...] = lax.dot_general(
            cb_ref[pl.ds(c * BN, BN), :], zb2, (((1,), (1,)), ((), ())),
            preferred_element_type=jnp.float32)
        r2c = r2_ref[pl.ds(c * BN, BN), :]
        d3 = ((r1v + r2c) - m_s[...]).reshape(rpc, 8, BM)

        lo = c * rpc
        for w, (wlo, whi) in enumerate(((0, RW0), (RW0, RW1), (RW1, NR))):
            slo, shi = max(lo, wlo), min(lo + rpc, whi)
            if slo >= shi:
                continue
            seg = d3[slo - lo:shi - lo]
            nr = shi - slo
            segmin = jnp.min(seg, axis=0)
            riota = lax.broadcasted_iota(jnp.int32, (nr, 8, BM), 0).astype(jnp.float32)
            cand = jnp.where(seg == segmin[None], riota, jnp.inf)
            segidx = jnp.min(cand, axis=0) + jnp.float32(slo)
            bv, bi = wins[w]
            take = (segmin < bv) | ((segmin == bv) & (segidx < bi))
            wins[w] = (jnp.where(take, segmin, bv), jnp.where(take, segidx, bi))

    s_iota = lax.broadcasted_iota(jnp.int32, (8, BM), 0).astype(jnp.float32)

    def smerge(vals, js):
        js = js * 8 + s_iota
        for h in (4, 2, 1):
            v1, v2 = vals[0:h], vals[h:2 * h]
            j1, j2 = js[0:h], js[h:2 * h]
            take = (v2 < v1) | ((v2 == v1) & (j2 < j1))
            vals = jnp.where(take, v2, v1)
            js = jnp.where(take, j2, j1)
        return vals, js

    (v0, i0) = smerge(*wins[0])
    (v1, i1) = smerge(*wins[1])
    (v2, i2) = smerge(*wins[2])

    accv = _bf16_rne(v0)
    acci = i0
    pickv = v0
    take = v1 < accv
    accv = jnp.where(take, _bf16_rne(v1), accv)
    acci = jnp.where(take, i1, acci)
    pickv = jnp.where(take, v1, pickv)
    take = v2 < accv
    acci = jnp.where(take, i2, acci)
    pickv = jnp.where(take, v2, pickv)

    idx_ref[...] = acci.astype(jnp.int32).reshape(BM)
    dpick_ref[...] = pickv.reshape(BM)


def _dist_argmin(r1, r2, z_flat, codebook):
    return pl.pallas_call(
        _dist_argmin_body,
        grid=(N_TOK // BM,),
        in_specs=[
            pl.BlockSpec((1, BM), lambda i: (0, i)),
            pl.BlockSpec((NUM_E, 1), lambda i: (0, 0)),
            pl.BlockSpec((BM, DIM), lambda i: (i, 0)),
            pl.BlockSpec((NUM_E, DIM), lambda i: (0, 0)),
        ],
        out_specs=[
            pl.BlockSpec((BM,), lambda i: (i,)),
            pl.BlockSpec((BM,), lambda i: (i,)),
        ],
        out_shape=[
            jax.ShapeDtypeStruct((N_TOK,), jnp.int32),
            jax.ShapeDtypeStruct((N_TOK,), jnp.float32),
        ],
        scratch_shapes=[pltpu.VMEM((BN, BM), jnp.float32),
                        pltpu.VMEM((BN, BM), jnp.float32)],
    )(r1, r2, z_flat, codebook)


_SC_CHUNK = 128


def _sc_gather(codebook, idx):
    info = plsc.get_sparse_core_info()
    nw = info.num_cores * info.num_subcores
    b_per_w = N_TOK // nw
    nchunk = b_per_w // _SC_CHUNK
    mesh = plsc.VectorSubcoreMesh(core_axis_name="c", subcore_axis_name="s")

    @functools.partial(
        pl.kernel, mesh=mesh,
        out_type=jax.ShapeDtypeStruct((N_TOK, DIM), jnp.float32),
        scratch_types=[
            pltpu.VMEM((_SC_CHUNK,), jnp.int32),
            pltpu.VMEM((_SC_CHUNK, DIM), jnp.float32),
            pltpu.SemaphoreType.DMA,
        ],
    )
    def gather(table_hbm, idx_hbm, out_hbm, idx_v, rows_v, sem):
        wid = lax.axis_index("s") * info.num_cores + lax.axis_index("c")
        base = wid * b_per_w
        for t in range(nchunk):
            off = base + t * _SC_CHUNK
            pltpu.sync_copy(idx_hbm.at[pl.ds(off, _SC_CHUNK)], idx_v)
            pltpu.async_copy(table_hbm.at[idx_v], rows_v, sem).wait()
            pltpu.sync_copy(rows_v, out_hbm.at[pl.ds(off, _SC_CHUNK)])

    return gather(codebook, idx)


def kernel(z, codebook):
    zz = z[0]
    z_flat = zz.reshape(-1, DIM)
    r1 = jnp.sum(zz ** 2, axis=-1).reshape(1, -1)
    r2 = jnp.sum(codebook ** 2, axis=1).reshape(-1, 1)
    idx, dpick = _dist_argmin(r1, r2, z_flat, codebook)
    zq = _sc_gather(codebook, idx)
    z_out = zq.reshape(zz.shape)
    vq_loss = 1.1 * (jnp.sum(dpick) / jnp.float32(z_flat.size))
    return (z_out, vq_loss)


# speedup vs baseline: 6.9549x; 1.0187x over previous
import functools

import jax
import jax.numpy as jnp
from jax import lax
from jax.experimental import pallas as pl
from jax.experimental.pallas import tpu as pltpu
from jax.experimental.pallas import tpu_sc as plsc

NUM_E = 8192
DIM = 256
N_TOK = 16384
BM = 2048
BN = 1024
RW0, RW1 = 342, 684
NR = NUM_E // 8


def _bf16_rne(x):
    b = lax.bitcast_convert_type(x, jnp.uint32)
    r = (b + jnp.uint32(0x7FFF) + ((b >> 16) & jnp.uint32(1))) & jnp.uint32(0xFFFF0000)
    return lax.bitcast_convert_type(r, jnp.float32)


def _dist_argmin_body(r1_ref, r2_ref, z_ref, cb_ref, idx_ref, dpick_ref, m_s0, m_s1):
    zb2 = z_ref[...] * 2.0
    r1v = r1_ref[...]

    inf8 = jnp.full((8, BM), jnp.inf, jnp.float32)
    zero8 = jnp.zeros((8, BM), jnp.float32)
    wins = [(inf8, zero8), (inf8, zero8), (inf8, zero8)]

    nchunks = NUM_E // BN
    rpc = BN // 8
    bufs = (m_s0, m_s1)

    for c in range(nchunks):
        m_s = bufs[c % 2]
        m_s[...] = lax.dot_general(
            cb_ref[pl.ds(c * BN, BN), :], zb2, (((1,), (1,)), ((), ())),
            preferred_element_type=jnp.float32)
        r2c = r2_ref[pl.ds(c * BN, BN), :]
        d3 = ((r1v + r2c) - m_s[...]).reshape(rpc, 8, BM)

        lo = c * rpc
        for w, (wlo, whi) in enumerate(((0, RW0), (RW0, RW1), (RW1, NR))):
            slo, shi = max(lo, wlo), min(lo + rpc, whi)
            if slo >= shi:
                continue
            seg = d3[slo - lo:shi - lo]
            nr = shi - slo
            segmin = jnp.min(seg, axis=0)
            riota = lax.broadcasted_iota(jnp.int32, (nr, 8, BM), 0).astype(jnp.float32)
            cand = jnp.where(seg == segmin[None], riota, jnp.inf)
            segidx = jnp.min(cand, axis=0) + jnp.float32(slo)
            bv, bi = wins[w]
            take = (segmin < bv) | ((segmin == bv) & (segidx < bi))
            wins[w] = (jnp.where(take, segmin, bv), jnp.where(take, segidx, bi))

    s_iota = lax.broadcasted_iota(jnp.int32, (8, BM), 0).astype(jnp.float32)

    def smerge(vals, js):
        js = js * 8 + s_iota
        for h in (4, 2, 1):
            v1, v2 = vals[0:h], vals[h:2 * h]
            j1, j2 = js[0:h], js[h:2 * h]
            take = (v2 < v1) | ((v2 == v1) & (j2 < j1))
            vals = jnp.where(take, v2, v1)
            js = jnp.where(take, j2, j1)
        return vals, js

    (v0, i0) = smerge(*wins[0])
    (v1, i1) = smerge(*wins[1])
    (v2, i2) = smerge(*wins[2])

    accv = _bf16_rne(v0)
    acci = i0
    pickv = v0
    take = v1 < accv
    accv = jnp.where(take, _bf16_rne(v1), accv)
    acci = jnp.where(take, i1, acci)
    pickv = jnp.where(take, v1, pickv)
    take = v2 < accv
    acci = jnp.where(take, i2, acci)
    pickv = jnp.where(take, v2, pickv)

    idx_ref[...] = acci.astype(jnp.int32).reshape(BM)
    dpick_ref[...] = pickv.reshape(BM)


def _dist_argmin(r1, r2, z_flat, codebook):
    return pl.pallas_call(
        _dist_argmin_body,
        grid=(N_TOK // BM,),
        in_specs=[
            pl.BlockSpec((1, BM), lambda i: (0, i)),
            pl.BlockSpec((NUM_E, 1), lambda i: (0, 0)),
            pl.BlockSpec((BM, DIM), lambda i: (i, 0)),
            pl.BlockSpec((NUM_E, DIM), lambda i: (0, 0)),
        ],
        out_specs=[
            pl.BlockSpec((BM,), lambda i: (i,)),
            pl.BlockSpec((BM,), lambda i: (i,)),
        ],
        out_shape=[
            jax.ShapeDtypeStruct((N_TOK,), jnp.int32),
            jax.ShapeDtypeStruct((N_TOK,), jnp.float32),
        ],
        scratch_shapes=[pltpu.VMEM((BN, BM), jnp.float32),
                        pltpu.VMEM((BN, BM), jnp.float32)],
    )(r1, r2, z_flat, codebook)


_SC_CHUNK = 128


def _sc_gather(codebook, idx):
    info = plsc.get_sparse_core_info()
    nw = info.num_cores * info.num_subcores
    b_per_w = N_TOK // nw
    nchunk = b_per_w // _SC_CHUNK
    mesh = plsc.VectorSubcoreMesh(core_axis_name="c", subcore_axis_name="s")

    @functools.partial(
        pl.kernel, mesh=mesh,
        out_type=jax.ShapeDtypeStruct((N_TOK, DIM), jnp.float32),
        scratch_types=[
            pltpu.VMEM((_SC_CHUNK,), jnp.int32),
            pltpu.VMEM((_SC_CHUNK, DIM), jnp.float32),
            pltpu.SemaphoreType.DMA,
        ],
    )
    def gather(table_hbm, idx_hbm, out_hbm, idx_v, rows_v, sem):
        wid = lax.axis_index("s") * info.num_cores + lax.axis_index("c")
        base = wid * b_per_w
        for t in range(nchunk):
            off = base + t * _SC_CHUNK
            pltpu.sync_copy(idx_hbm.at[pl.ds(off, _SC_CHUNK)], idx_v)
            pltpu.async_copy(table_hbm.at[idx_v], rows_v, sem).wait()
            pltpu.sync_copy(rows_v, out_hbm.at[pl.ds(off, _SC_CHUNK)])

    return gather(codebook, idx)


def kernel(z, codebook):
    zz = z[0]
    z_flat = zz.reshape(-1, DIM)
    r1 = jnp.sum(zz ** 2, axis=-1).reshape(1, -1)
    r2 = jnp.sum(codebook ** 2, axis=1).reshape(-1, 1)
    idx, dpick = _dist_argmin(r1, r2, z_flat, codebook)
    zq = _sc_gather(codebook, idx)
    z_out = zq.reshape(zz.shape)
    vq_loss = 1.1 * (jnp.sum(dpick) / jnp.float32(z_flat.size))
    return (z_out, vq_loss)
